# Initial kernel scaffold; baseline (speedup 1.0000x reference)
import jax
import jax.numpy as jnp
from jax.experimental import pallas as pl


def kernel(x, table):
    raise NotImplementedError("write your pallas kernel here")


# speedup vs baseline: 1.0321x; 1.0321x over previous
import math

import jax
import jax.numpy as jnp
import numpy as np
from jax import lax
from jax.experimental import pallas as pl
from jax.experimental.pallas import tpu as pltpu
from jax.experimental.pallas import tpu_sc as plsc

DIM = 128
SEQ = 2048
BATCH = 4
SCALE = np.float32(math.sqrt(DIM))

NC = 2
NS = 16
NW = NC * NS
B = BATCH * SEQ
B_PER_W = B // NW
CHUNK = 128
NCHUNK = B_PER_W // CHUNK
LANES = 16


def _pe_table():
    position = np.arange(SEQ, dtype=np.float32)[:, None]
    div_term = np.exp(
        np.arange(0, DIM, 2, dtype=np.float32) * (-math.log(10000.0) / DIM))
    pe = np.zeros((SEQ, DIM), dtype=np.float32)
    pe[:, 0::2] = np.sin(position * div_term)
    pe[:, 1::2] = np.cos(position * div_term)
    return pe


_PE = _pe_table()


def _embed_body(idx_hbm, table_hbm, pe_hbm, out_hbm,
                idx0, idx1, rows0, rows1, pe_v, sem0, sem1):
    wid = lax.axis_index("s") * NC + lax.axis_index("c")
    base = wid * B_PER_W
    pbase = lax.rem(base, SEQ)

    pltpu.sync_copy(idx_hbm.at[pl.ds(base, CHUNK)], idx0)
    pltpu.sync_copy(idx_hbm.at[pl.ds(base + CHUNK, CHUNK)], idx1)
    cp0 = pltpu.async_copy(table_hbm.at[idx0], rows0, sem0)
    cp1 = pltpu.async_copy(table_hbm.at[idx1], rows1, sem1)
    pltpu.sync_copy(pe_hbm.at[pl.ds(pbase, B_PER_W)], pe_v)

    cp0.wait()

    def row0(i, carry):
        for j in range(DIM // LANES):
            sl = pl.ds(j * LANES, LANES)
            rows0[i, sl] = rows0[i, sl] * SCALE + pe_v[i, sl]
        return carry

    lax.fori_loop(0, CHUNK, row0, 0)
    pltpu.sync_copy(rows0, out_hbm.at[pl.ds(base, CHUNK)])

    cp1.wait()

    def row1(i, carry):
        for j in range(DIM // LANES):
            sl = pl.ds(j * LANES, LANES)
            rows1[i, sl] = rows1[i, sl] * SCALE + pe_v[CHUNK + i, sl]
        return carry

    lax.fori_loop(0, CHUNK, row1, 0)
    pltpu.sync_copy(rows1, out_hbm.at[pl.ds(base + CHUNK, CHUNK)])


def kernel(x, table):
    xf = x.reshape(-1)
    pe = jnp.asarray(_PE)
    call = pl.kernel(
        _embed_body,
        out_type=jax.ShapeDtypeStruct((B, DIM), jnp.float32),
        mesh=plsc.VectorSubcoreMesh(core_axis_name="c", subcore_axis_name="s"),
        scratch_types=[
            pltpu.VMEM((CHUNK,), jnp.int32),
            pltpu.VMEM((CHUNK,), jnp.int32),
            pltpu.VMEM((CHUNK, DIM), jnp.float32),
            pltpu.VMEM((CHUNK, DIM), jnp.float32),
            pltpu.VMEM((B_PER_W, DIM), jnp.float32),
            pltpu.SemaphoreType.DMA,
            pltpu.SemaphoreType.DMA,
        ],
    )
    out = call(xf, table, pe)
    return out.reshape(BATCH, SEQ, DIM)


# speedup vs baseline: 1.0742x; 1.0408x over previous
import math

import jax
import jax.numpy as jnp
import numpy as np
from jax import lax
from jax.experimental import pallas as pl
from jax.experimental.pallas import tpu as pltpu
from jax.experimental.pallas import tpu_sc as plsc

DIM = 128
SEQ = 2048
BATCH = 4
SCALE = np.float32(math.sqrt(DIM))

NC = 2
NS = 16
NW = NC * NS
B = BATCH * SEQ
B_PER_W = B // NW
CHUNK = 128
NCHUNK = B_PER_W // CHUNK
LANES = 16


def _pe_table():
    position = np.arange(SEQ, dtype=np.float32)[:, None]
    div_term = np.exp(
        np.arange(0, DIM, 2, dtype=np.float32) * (-math.log(10000.0) / DIM))
    pe = np.zeros((SEQ, DIM), dtype=np.float32)
    pe[:, 0::2] = np.sin(position * div_term)
    pe[:, 1::2] = np.cos(position * div_term)
    return pe


_PE = _pe_table()


def _embed_body(idx_hbm, table_hbm, pe_hbm, out_hbm,
                idx_v, rows0, rows1, buf, sem0, sem1, sem_pe, sem_o0, sem_o1):
    wid = lax.axis_index("s") * NC + lax.axis_index("c")
    base = wid * B_PER_W
    pbase = lax.rem(base, SEQ)

    pltpu.sync_copy(idx_hbm.at[pl.ds(NCHUNK * wid, NCHUNK)], idx_v)
    cp0 = pltpu.async_copy(table_hbm.at[idx_v.at[0]], rows0, sem0)
    cp1 = pltpu.async_copy(table_hbm.at[idx_v.at[1]], rows1, sem1)
    cpe = pltpu.async_copy(pe_hbm.at[pl.ds(pbase, B_PER_W)], buf, sem_pe)

    cpe.wait()
    cp0.wait()

    def row0(i, carry):
        for j in range(DIM // LANES):
            sl = pl.ds(j * LANES, LANES)
            plsc.addupdate(buf.at[i, sl], rows0[i, sl] * SCALE)
        return carry

    lax.fori_loop(0, CHUNK, row0, 0)
    co0 = pltpu.async_copy(buf.at[pl.ds(0, CHUNK)],
                           out_hbm.at[pl.ds(base, CHUNK)], sem_o0)

    cp1.wait()

    def row1(i, carry):
        for j in range(DIM // LANES):
            sl = pl.ds(j * LANES, LANES)
            plsc.addupdate(buf.at[CHUNK + i, sl], rows1[i, sl] * SCALE)
        return carry

    lax.fori_loop(0, CHUNK, row1, 0)
    co1 = pltpu.async_copy(buf.at[pl.ds(CHUNK, CHUNK)],
                           out_hbm.at[pl.ds(base + CHUNK, CHUNK)], sem_o1)
    co0.wait()
    co1.wait()


def kernel(x, table):
    xf = x.reshape(B // CHUNK, CHUNK)
    pe = jnp.asarray(_PE)
    call = pl.kernel(
        _embed_body,
        out_type=jax.ShapeDtypeStruct((B, DIM), jnp.float32),
        mesh=plsc.VectorSubcoreMesh(core_axis_name="c", subcore_axis_name="s"),
        scratch_types=[
            pltpu.VMEM((NCHUNK, CHUNK), jnp.int32),
            pltpu.VMEM((CHUNK, DIM), jnp.float32),
            pltpu.VMEM((CHUNK, DIM), jnp.float32),
            pltpu.VMEM((B_PER_W, DIM), jnp.float32),
            pltpu.SemaphoreType.DMA,
            pltpu.SemaphoreType.DMA,
            pltpu.SemaphoreType.DMA,
            pltpu.SemaphoreType.DMA,
            pltpu.SemaphoreType.DMA,
        ],
    )
    out = call(xf, table, pe)
    return out.reshape(BATCH, SEQ, DIM)
